# baseline (speedup 1.0000x reference)
import functools

import jax
import jax.numpy as jnp
from jax.experimental import pallas as pl
from jax.experimental.pallas import tpu as pltpu


def _fused_body(out_ref, out_b_ref, tgt_ref, tgt_b_ref,
                w1_ref, b1_ref, w2_ref, b2_ref, bins_ref,
                part_ref, acc_ref, *, img_per_step, c, inv_hw):
    i = pl.program_id(1)

    @pl.when(i == 0)
    def _init():
        acc_ref[0, 0] = 0.0
        acc_ref[0, 1] = 0.0

    def score_sum(x):
        pooled = jnp.sum(x.reshape(img_per_step, c, x.shape[-1]),
                         axis=2) * inv_hw
        h = jnp.dot(pooled, w1_ref[...],
                    preferred_element_type=jnp.float32) + b1_ref[...]
        h = jnp.maximum(h, 0.0)
        logits = jnp.dot(h, w2_ref[...],
                         preferred_element_type=jnp.float32) + b2_ref[...]
        m = jnp.max(logits, axis=-1, keepdims=True)
        e = jnp.exp(logits - m)
        p = e / jnp.sum(e, axis=-1, keepdims=True)
        return jnp.sum(p * bins_ref[...])

    acc_ref[0, 0] += score_sum(out_ref[...]) + score_sum(out_b_ref[...])
    acc_ref[0, 1] += score_sum(tgt_ref[...]) + score_sum(tgt_b_ref[...])

    @pl.when(i == pl.num_programs(1) - 1)
    def _finalize():
        part_ref[0, 0, 0] = acc_ref[0, 0]
        part_ref[0, 0, 1] = acc_ref[0, 1]


def kernel(out_img, tgt_img, w1, b1, w2, b2, bins):
    N, C, H, W = out_img.shape
    HW = H * W
    NC = N * C
    itemsize = jnp.dtype(out_img.dtype).itemsize

    img_per_step = 4
    while N % (4 * img_per_step) != 0 and img_per_step > 1:
        img_per_step //= 2
    blk = img_per_step * C
    if N % (4 * img_per_step) == 0:
        grid = (2, N // (4 * img_per_step))
    else:
        grid = (1, N // (2 * img_per_step))
    spc = grid[1]

    out2d = out_img.reshape(NC, HW)
    tgt2d = tgt_img.reshape(NC, HW)

    spec_a = pl.BlockSpec((blk, HW), lambda cc, i: (cc * 2 * spc + i, 0))
    spec_b = pl.BlockSpec((blk, HW), lambda cc, i: (cc * 2 * spc + spc + i, 0))
    full = lambda s: pl.BlockSpec(s, lambda cc, i: tuple(0 for _ in s))

    bytes_streamed = 2 * NC * HW * itemsize
    parts = pl.pallas_call(
        functools.partial(_fused_body, img_per_step=img_per_step, c=C,
                          inv_hw=1.0 / float(HW)),
        out_shape=jax.ShapeDtypeStruct((grid[0], 1, 2), jnp.float32),
        grid=grid,
        in_specs=[spec_a, spec_b, spec_a, spec_b,
                  full(w1.shape), full(b1.shape), full(w2.shape),
                  full(b2.shape), full(bins.shape)],
        out_specs=pl.BlockSpec((1, 1, 2), lambda cc, i: (cc, 0, 0),
                               memory_space=pltpu.MemorySpace.SMEM),
        scratch_shapes=[pltpu.SMEM((1, 2), jnp.float32)],
        compiler_params=pltpu.CompilerParams(
            dimension_semantics=("parallel", "arbitrary"),
            vmem_limit_bytes=64 * 1024 * 1024),
        cost_estimate=pl.CostEstimate(
            flops=2 * NC * HW + 4 * N * C * w1.shape[1],
            transcendentals=2 * N * w2.shape[1],
            bytes_accessed=bytes_streamed),
    )(out2d, out2d, tgt2d, tgt2d, w1, b1, w2, b2, bins)

    return jnp.abs(jnp.sum(parts[:, 0, 1]) -
                   jnp.sum(parts[:, 0, 0])) * (1.0 / float(N))
